# Initial kernel scaffold; baseline (speedup 1.0000x reference)
import jax
import jax.numpy as jnp
from jax.experimental import pallas as pl


def kernel(xq, xt, edge_index_q, edge_index_t, norm_q, norm_t, u2v, node_mask, Wq0, Wq1, Wq2, Wt0, Wt1, Wt2, Wint_q, Wint_t):
    raise NotImplementedError("write your pallas kernel here")


# speedup vs baseline: 2.4619x; 2.4619x over previous
import jax
import jax.numpy as jnp
from jax import lax
from jax.experimental import pallas as pl
from jax.experimental.pallas import tpu as pltpu
from jax.experimental.pallas import tpu_sc as plsc

N_SUBCORES = 16
CHUNK = 128


def _half_rows(N):
  return -(-N // (2 * N_SUBCORES * 8)) * N_SUBCORES * 8


def _make_prop(N, D, chunks, with_norm):
  nh = _half_rows(N)
  rows_per_tile = nh // N_SUBCORES
  f32 = jnp.float32
  mesh = plsc.VectorSubcoreMesh(core_axis_name="c", subcore_axis_name="s")

  scratch = [
      pltpu.VMEM((chunks, CHUNK), jnp.int32),
      pltpu.VMEM((chunks, CHUNK), jnp.int32),
      pltpu.VMEM((CHUNK,), jnp.int32),
      pltpu.VMEM((CHUNK, D), f32),
      pltpu.VMEM_SHARED((nh + 8, D), f32),
      pltpu.SemaphoreType.DMA,
  ]
  if with_norm:
    scratch.append(pltpu.VMEM((chunks, CHUNK), f32))

  def body(*refs):
    if with_norm:
      (x0, s0, d0, n0, x1, s1, d1, n1, zeros,
       out0, out1, sidx, didx, dadj, rows, acc, sem, nrm) = refs
    else:
      (x0, s0, d0, x1, s1, d1, zeros,
       out0, out1, sidx, didx, dadj, rows, acc, sem) = refs
      n0 = n1 = nrm = None
    c = lax.axis_index("c")
    s = lax.axis_index("s")
    row0 = s * rows_per_tile

    def run_graph(x, sh, dh, nhh, out):
      pltpu.sync_copy(sh.at[s], sidx)
      pltpu.sync_copy(dh.at[s], didx)
      if with_norm:
        pltpu.sync_copy(nhh.at[s], nrm)

      for p in range(2):
        lo = p * nh
        pltpu.sync_copy(zeros, acc.at[pl.ds(row0, rows_per_tile)])
        plsc.subcore_barrier()

        def chunk_body(k, carry):
          pltpu.async_copy(x.at[sidx.at[k]], rows, sem).wait()
          for l in range(CHUNK // 16):
            sl = pl.ds(l * 16, 16)
            d = didx[k, sl] - lo
            ok = (d >= 0) & (d < nh)
            dadj[sl] = jnp.where(ok, d, nh)
          if with_norm:
            def scale(e16, cc):
              nv16 = nrm[k, pl.ds(e16 * 16, 16)]
              for l in range(16):
                nvec = jnp.full((16,), nv16[l], f32)
                e = e16 * 16 + l
                for j in range(D // 16):
                  sl = pl.ds(j * 16, 16)
                  rows[e, sl] = rows[e, sl] * nvec
              return cc
            lax.fori_loop(0, CHUNK // 16, scale, 0)
          pltpu.sync_copy(rows, acc.at[dadj], add=True)
          return carry

        lax.fori_loop(0, chunks, chunk_body, 0)
        plsc.subcore_barrier()
        pltpu.sync_copy(acc.at[pl.ds(row0, rows_per_tile)],
                        out.at[pl.ds(lo + row0, rows_per_tile)])

    @pl.when(c == 0)
    def _():
      run_graph(x0, s0, d0, n0, out0)

    @pl.when(c == 1)
    def _():
      run_graph(x1, s1, d1, n1, out1)

  out_type = [jax.ShapeDtypeStruct((2 * nh, D), f32)] * 2
  return pl.kernel(body, out_type=out_type, mesh=mesh, scratch_types=scratch)


def _dense_layer(aq, at, Wq, Wt, mq, mt, apply_elu):
  N, D = mq.shape
  R = 1000
  f32 = jnp.float32

  def body(aq_r, at_r, wq_r, wt_r, mq_r, mt_r, xq_o, xt_o, mq_o, mt_o):
    xq = jnp.dot(aq_r[...], wq_r[...], preferred_element_type=f32)
    xt = jnp.dot(at_r[...], wt_r[...], preferred_element_type=f32)
    if apply_elu:
      xq = jnp.where(xq > 0, xq, jnp.exp(xq) - 1.0)
      xt = jnp.where(xt > 0, xt, jnp.exp(xt) - 1.0)
    xq_o[...] = xq
    xt_o[...] = xt
    mq_o[...] = jnp.maximum(mq_r[...], xq)
    mt_o[...] = jnp.maximum(mt_r[...], xt)

  row = pl.BlockSpec((R, D), lambda i: (i, 0))
  w = pl.BlockSpec((D, D), lambda i: (0, 0))
  return pl.pallas_call(
      body,
      grid=(N // R,),
      in_specs=[row, row, w, w, row, row],
      out_specs=[row, row, row, row],
      out_shape=[jax.ShapeDtypeStruct((N, D), f32)] * 4,
  )(aq, at, Wq, Wt, mq, mt)


def _final_combine(Xq, Xt, cq, ct, Wiq, Wit, mask):
  N, D = Xq.shape
  R = 1000
  f32 = jnp.float32

  def body(xq_r, xt_r, cq_r, ct_r, wq_r, wt_r, m_r, oq, ot):
    oq[...] = xq_r[...] + m_r[...] * jnp.dot(
        cq_r[...], wq_r[...], preferred_element_type=f32)
    ot[...] = xt_r[...] + jnp.dot(
        ct_r[...], wt_r[...], preferred_element_type=f32)

  row = pl.BlockSpec((R, D), lambda i: (i, 0))
  w = pl.BlockSpec((D, D), lambda i: (0, 0))
  m = pl.BlockSpec((R, 1), lambda i: (i, 0))
  return pl.pallas_call(
      body,
      grid=(N // R,),
      in_specs=[row, row, row, row, w, w, m],
      out_specs=[row, row],
      out_shape=[jax.ShapeDtypeStruct((N, D), f32)] * 2,
  )(Xq, Xt, cq, ct, Wiq, Wit, mask)


def kernel(xq, xt, edge_index_q, edge_index_t, norm_q, norm_t, u2v, node_mask,
           Wq0, Wq1, Wq2, Wt0, Wt1, Wt2, Wint_q, Wint_t):
  N, D = xq.shape
  E = edge_index_q.shape[1]
  chunks = -(-(E // N_SUBCORES) // CHUNK)
  e_pad = N_SUBCORES * chunks * CHUNK
  f32 = jnp.float32

  def prep(gather_idx, scatter_idx, nrm):
    pad = e_pad - E
    g = jnp.pad(gather_idx, (0, pad)).reshape(N_SUBCORES, chunks, CHUNK)
    sc = jnp.pad(scatter_idx, (0, pad), constant_values=N).reshape(
        N_SUBCORES, chunks, CHUNK)
    if nrm is None:
      return g, sc, None
    return g, sc, jnp.pad(nrm, (0, pad)).reshape(N_SUBCORES, chunks, CHUNK)

  sq, dq, nq = prep(edge_index_q[0], edge_index_q[1], norm_q)
  st, dt, nt = prep(edge_index_t[0], edge_index_t[1], norm_t)
  gv, su, _ = prep(u2v[1], u2v[0], None)
  gu, sv, _ = prep(u2v[0], u2v[1], None)

  zeros = jnp.zeros((_half_rows(N) // N_SUBCORES, D), f32)
  prop_n = _make_prop(N, D, chunks, with_norm=True)
  prop_x = _make_prop(N, D, chunks, with_norm=False)

  Wq = [Wq0, Wq1, Wq2]
  Wt = [Wt0, Wt1, Wt2]
  x_q, x_t = xq, xt
  mq, mt = xq, xt
  for i in range(3):
    aq, at = prop_n(x_q, sq, dq, nq, x_t, st, dt, nt, zeros)
    x_q, x_t, mq, mt = _dense_layer(aq, at, Wq[i], Wt[i], mq, mt, i < 2)

  cq, ct = prop_x(mt, gv, su, mq, gu, sv, zeros)
  return _final_combine(mq, mt, cq, ct, Wint_q, Wint_t,
                        node_mask.reshape(N, 1))
